# baseline (speedup 1.0000x reference)
import functools

import jax
import jax.numpy as jnp
from jax import lax
from jax.experimental import pallas as pl
from jax.experimental.pallas import tpu as pltpu
from jax.experimental.pallas import tpu_sc as plsc

N = 10000
E = 320000
D_IN = 128
D_HID = 128
NA = 8
NAP = 16

NC = 2
NS = 16
NW = NC * NS
EPT = E // NW
EPT_PAD = 10080
PAD_PK = 0x0FFFFFFF
VBOUND = N * 16384
NP = 10240
RPS = NP // NS

ROWS_BLK = 1000
GRID = N // ROWS_BLK


def _mm2_body(x_ref, wl_ref, wr_ref, xl_ref, xr_ref):
    xb = x_ref[...]
    xl_ref[...] = jnp.dot(xb, wl_ref[...], preferred_element_type=jnp.float32)
    xr_ref[...] = jnp.dot(xb, wr_ref[...], preferred_element_type=jnp.float32)


def _mm2(x, wl, wr):
    d = wl.shape[1]
    return pl.pallas_call(
        _mm2_body,
        grid=(GRID,),
        in_specs=[
            pl.BlockSpec((ROWS_BLK, D_IN), lambda i: (i, 0)),
            pl.BlockSpec((D_IN, d), lambda i: (0, 0)),
            pl.BlockSpec((D_IN, d), lambda i: (0, 0)),
        ],
        out_specs=[
            pl.BlockSpec((ROWS_BLK, d), lambda i: (i, 0)),
            pl.BlockSpec((ROWS_BLK, d), lambda i: (i, 0)),
        ],
        out_shape=[
            jax.ShapeDtypeStruct((N, d), jnp.float32),
            jax.ShapeDtypeStruct((N, d), jnp.float32),
        ],
    )(x, wl, wr)


def _edge_body(d, ch, n, *refs):
    (xl_hbm, xr_hbm, edge_hbm, att_hbm, acc_hbm, den_hbm, pk_v) = refs[:7]
    si = refs[7:7 + n]
    di = refs[7 + n:7 + 2 * n]
    ab = refs[7 + 2 * n:7 + 3 * n]
    bl = refs[7 + 3 * n:7 + 4 * n]
    br = refs[7 + 4 * n:7 + 5 * n]
    wb = refs[7 + 5 * n:7 + 6 * n]
    att_v = refs[7 + 6 * n]
    acc_sh, den_sh = refs[8 + 6 * n:10 + 6 * n]
    gs = refs[10 + 6 * n:10 + 7 * n]
    ss = refs[10 + 7 * n:10 + 8 * n]
    c = lax.axis_index("c")
    s = lax.axis_index("s")
    wid = c * NS + s
    nk = d // 16
    ngr = ch // 16
    nchunk = EPT_PAD // ch
    z16 = jnp.zeros((16,), jnp.float32)
    lanes = lax.iota(jnp.int32, 16)

    pltpu.sync_copy(edge_hbm.at[wid], pk_v)
    pltpu.sync_copy(att_hbm, att_v)

    for e in range(16):
        for k in range(nk):
            bl[0][e, pl.ds(k * 16, 16)] = z16
    wb[0][pl.ds(0, 16)] = z16

    def zcopy(i, carry):
        pltpu.sync_copy(bl[0].at[pl.ds(0, 16)],
                        acc_sh.at[pl.ds(s * RPS + i * 16, 16)])
        pltpu.sync_copy(wb[0].at[pl.ds(0, 16)],
                        den_sh.at[pl.ds(s * RPS + i * 16, 16)])
        return carry

    lax.fori_loop(0, RPS // 16, zcopy, 0, unroll=False)
    plsc.subcore_barrier()

    def decode(slot, ci):
        for g in range(ngr):
            pk = pk_v[pl.ds(ci * ch + g * 16, 16)]
            si[slot][pl.ds(g * 16, 16)] = jnp.minimum(
                lax.shift_right_logical(pk, 14), N - 1)
            di[slot][pl.ds(g * 16, 16)] = jnp.minimum(pk & 0x3FFF, N - 1)
            ab[slot][pl.ds(g * 16, 16)] = jnp.where(
                pk >= VBOUND, -1e30, 0.0)

    def fire_gather(slot):
        pltpu.async_copy(xl_hbm.at[si[slot]], bl[slot], gs[slot])
        pltpu.async_copy(xr_hbm.at[di[slot]], br[slot], gs[slot])

    def drain_gather(slot):
        pltpu.make_async_copy(xl_hbm.at[pl.ds(0, ch)], bl[slot], gs[slot]).wait()
        pltpu.make_async_copy(xl_hbm.at[pl.ds(0, ch)], br[slot], gs[slot]).wait()

    def fire_scatter(slot):
        pltpu.async_copy(bl[slot], acc_sh.at[di[slot]], ss[slot], add=True)
        pltpu.async_copy(wb[slot], den_sh.at[di[slot]], ss[slot], add=True)

    def drain_scatter(slot):
        pltpu.make_async_copy(xl_hbm.at[pl.ds(0, ch)], bl[slot], ss[slot]).wait()
        pltpu.make_async_copy(den_hbm.at[0, pl.ds(0, ch)], wb[slot], ss[slot]).wait()

    def compute(slot):
        blb = bl[slot]
        brb = br[slot]

        def grp(g2, carry2):
            base = g2 * 16
            ab16 = ab[slot][pl.ds(base, 16)]
            wden = jnp.zeros((16,), jnp.float32)
            for e in range(16):
                row = base + e
                part = jnp.zeros((16,), jnp.float32)
                lregs = []
                for k in range(nk):
                    l = blb[row, pl.ds(k * 16, 16)]
                    r = brb[row, pl.ds(k * 16, 16)]
                    lregs.append(l)
                    v = l + r
                    v = jnp.maximum(v, 0.2 * v)
                    part = part + v * att_v[pl.ds(k * 16, 16)]
                for sh in (8, 4, 2, 1):
                    part = part + jnp.take_along_axis(part, lanes ^ sh, axis=0)
                e16 = jnp.full((16,), e, jnp.int32)
                w = jnp.exp(part + jnp.take_along_axis(ab16, e16, axis=0))
                for k in range(nk):
                    blb[row, pl.ds(k * 16, 16)] = lregs[k] * w
                wden = jnp.where(lanes == e, w, wden)
            wb[slot][pl.ds(base, 16)] = wden
            return carry2

        lax.fori_loop(0, ngr, grp, 0, unroll=False)

    for f in range(n - 1):
        decode(f, f)
        fire_gather(f)

    def ring(g, carry):
        for b in range(n):
            ci = g * n + b
            q = (b + n - 1) % n

            @pl.when(ci >= 1)
            def _():
                drain_scatter(q)

            @pl.when(ci + n - 1 < nchunk)
            def _():
                decode(q, ci + n - 1)
                fire_gather(q)

            drain_gather(b)
            compute(b)
            fire_scatter(b)
        return carry

    lax.fori_loop(0, nchunk // n, ring, 0, unroll=False)
    drain_scatter(n - 1)
    plsc.subcore_barrier()
    pltpu.sync_copy(acc_sh.at[pl.ds(s * RPS, RPS)],
                    acc_hbm.at[c, pl.ds(s * RPS, RPS)])
    pltpu.sync_copy(den_sh.at[pl.ds(s * RPS, RPS)],
                    den_hbm.at[c, pl.ds(s * RPS, RPS)])


def _edge_phase(d, untiled, ch, n, xl, xr, edge2, att):
    assert EPT_PAD % ch == 0 and (EPT_PAD // ch) % n == 0 and ch % 16 == 0
    mesh = plsc.VectorSubcoreMesh(core_axis_name="c", subcore_axis_name="s")
    scratch = (
        [pltpu.VMEM((EPT_PAD,), jnp.int32)]
        + [pltpu.VMEM((ch,), jnp.int32)] * (2 * n)
        + [pltpu.VMEM((ch,), jnp.float32)] * n
        + [pltpu.VMEM((ch, d), jnp.float32)] * (2 * n)
        + [pltpu.VMEM((ch,), jnp.float32)] * n
        + [pltpu.VMEM((d,), jnp.float32)]
        + [
            pltpu.VMEM_SHARED((NP, d), jnp.float32),
            pltpu.VMEM_SHARED((NP,), jnp.float32),
        ]
        + [pltpu.SemaphoreType.DMA] * (2 * n)
    )
    params = pltpu.CompilerParams(use_tc_tiling_on_sc=False) if untiled else None
    kfn = pl.kernel(
        functools.partial(_edge_body, d, ch, n),
        out_type=[
            jax.ShapeDtypeStruct((NC, NP, d), jnp.float32),
            jax.ShapeDtypeStruct((NC, NP), jnp.float32),
        ],
        mesh=mesh,
        scratch_types=scratch,
        compiler_params=params,
    )
    return kfn(xl, xr, edge2, att)


def _mid_body(acc_ref, den_ref, b_ref, wl_ref, wr_ref, xl_ref, xr_ref):
    acc = acc_ref[0] + acc_ref[1]
    den = den_ref[0] + den_ref[1] + 1e-16
    h = acc / den + b_ref[...]
    h = jnp.where(h > 0, h, jnp.exp(jnp.minimum(h, 0.0)) - 1.0)
    xl_ref[...] = jnp.dot(h, wl_ref[...], preferred_element_type=jnp.float32)
    xr_ref[...] = jnp.dot(h, wr_ref[...], preferred_element_type=jnp.float32)


def _mid(acc, den3, b1, wl2p, wr2p):
    return pl.pallas_call(
        _mid_body,
        grid=(GRID,),
        in_specs=[
            pl.BlockSpec((NC, ROWS_BLK, D_HID), lambda i: (0, i, 0)),
            pl.BlockSpec((NC, ROWS_BLK, 1), lambda i: (0, i, 0)),
            pl.BlockSpec((1, D_HID), lambda i: (0, 0)),
            pl.BlockSpec((D_HID, NAP), lambda i: (0, 0)),
            pl.BlockSpec((D_HID, NAP), lambda i: (0, 0)),
        ],
        out_specs=[
            pl.BlockSpec((ROWS_BLK, NAP), lambda i: (i, 0)),
            pl.BlockSpec((ROWS_BLK, NAP), lambda i: (i, 0)),
        ],
        out_shape=[
            jax.ShapeDtypeStruct((NP, NAP), jnp.float32),
            jax.ShapeDtypeStruct((NP, NAP), jnp.float32),
        ],
    )(acc, den3, b1.reshape(1, D_HID), wl2p, wr2p)


def _fin_body(acc_ref, den_ref, b_ref, out_ref):
    acc = acc_ref[0] + acc_ref[1]
    den = den_ref[0] + den_ref[1] + 1e-16
    logits = acc / den + b_ref[...]
    lane = lax.broadcasted_iota(jnp.int32, (ROWS_BLK, NAP), 1)
    valid = lane < NA
    neg = jnp.where(valid, logits, -jnp.inf)
    m = jnp.max(neg, axis=1, keepdims=True)
    ex = jnp.where(valid, jnp.exp(logits - m), 0.0)
    se = jnp.sum(ex, axis=1, keepdims=True)
    out_ref[...] = logits - m - jnp.log(se)


def _fin(acc, den3, b2p):
    return pl.pallas_call(
        _fin_body,
        grid=(GRID,),
        in_specs=[
            pl.BlockSpec((NC, ROWS_BLK, NAP), lambda i: (0, i, 0)),
            pl.BlockSpec((NC, ROWS_BLK, 1), lambda i: (0, i, 0)),
            pl.BlockSpec((1, NAP), lambda i: (0, 0)),
        ],
        out_specs=pl.BlockSpec((ROWS_BLK, NAP), lambda i: (i, 0)),
        out_shape=jax.ShapeDtypeStruct((N, NAP), jnp.float32),
    )(acc, den3, b2p.reshape(1, NAP))


@jax.jit
def kernel(x, edge_index, Wl1, Wr1, att1, b1, Wl2, Wr2, att2, b2):
    packed = edge_index[0] * 16384 + edge_index[1]
    pad = jnp.full((NW, EPT_PAD - EPT), PAD_PK, jnp.int32)
    edge2 = jnp.concatenate([packed.reshape(NW, EPT), pad], axis=1)
    wl2p = jnp.pad(Wl2, ((0, 0), (0, NAP - NA)))
    wr2p = jnp.pad(Wr2, ((0, 0), (0, NAP - NA)))
    att2p = jnp.pad(att2, (0, NAP - NA))
    b2p = jnp.pad(b2, (0, NAP - NA))

    xl1, xr1 = _mm2(x, Wl1, Wr1)
    acc1, den1 = _edge_phase(D_HID, False, 32, 3, xl1, xr1, edge2, att1)
    xl2, xr2 = _mid(acc1, den1.reshape(NC, NP, 1), b1, wl2p, wr2p)
    acc2, den2 = _edge_phase(NAP, True, 112, 6, xl2, xr2, edge2, att2p)
    out = _fin(acc2, den2.reshape(NC, NP, 1), b2p)
    return out[:, :NA]


# speedup vs baseline: 1.7917x; 1.7917x over previous
import functools

import jax
import jax.numpy as jnp
from jax import lax
from jax.experimental import pallas as pl
from jax.experimental.pallas import tpu as pltpu
from jax.experimental.pallas import tpu_sc as plsc

N = 10000
E = 320000
D_IN = 128
D_HID = 128
NA = 8
NAP = 16

NC = 2
NS = 16
NW = NC * NS
EPT = E // NW
EPT_PAD = 10080
PAD_PK = 0x0FFFFFFF
VBOUND = N * 16384
NP = 10240
RPS = NP // NS

ROWS_BLK = 1000
GRID = N // ROWS_BLK


def _mm2_body(x_ref, wl_ref, wr_ref, xl_ref, xr_ref):
    xb = x_ref[...]
    xl_ref[...] = jnp.dot(xb, wl_ref[...], preferred_element_type=jnp.float32)
    xr_ref[...] = jnp.dot(xb, wr_ref[...], preferred_element_type=jnp.float32)


def _mm2(x, wl, wr):
    d = wl.shape[1]
    return pl.pallas_call(
        _mm2_body,
        grid=(GRID,),
        in_specs=[
            pl.BlockSpec((ROWS_BLK, D_IN), lambda i: (i, 0)),
            pl.BlockSpec((D_IN, d), lambda i: (0, 0)),
            pl.BlockSpec((D_IN, d), lambda i: (0, 0)),
        ],
        out_specs=[
            pl.BlockSpec((ROWS_BLK, d), lambda i: (i, 0)),
            pl.BlockSpec((ROWS_BLK, d), lambda i: (i, 0)),
        ],
        out_shape=[
            jax.ShapeDtypeStruct((N, d), jnp.float32),
            jax.ShapeDtypeStruct((N, d), jnp.float32),
        ],
    )(x, wl, wr)


def _edge_body(d, ch, n, *refs):
    (xl_hbm, xr_hbm, edge_hbm, att_hbm, acc_hbm, den_hbm, pk_v) = refs[:7]
    si = refs[7:7 + n]
    di = refs[7 + n:7 + 2 * n]
    ab = refs[7 + 2 * n:7 + 3 * n]
    bl = refs[7 + 3 * n:7 + 4 * n]
    br = refs[7 + 4 * n:7 + 5 * n]
    wb = refs[7 + 5 * n:7 + 6 * n]
    att_v = refs[7 + 6 * n]
    acc_sh, den_sh = refs[8 + 6 * n:10 + 6 * n]
    gs = refs[10 + 6 * n:10 + 7 * n]
    ss = refs[10 + 7 * n:10 + 8 * n]
    c = lax.axis_index("c")
    s = lax.axis_index("s")
    wid = c * NS + s
    nk = d // 16
    ngr = ch // 16
    nchunk = EPT_PAD // ch
    z16 = jnp.zeros((16,), jnp.float32)
    lanes = lax.iota(jnp.int32, 16)

    pltpu.sync_copy(edge_hbm.at[wid], pk_v)
    pltpu.sync_copy(att_hbm, att_v)

    for e in range(16):
        for k in range(nk):
            bl[0][e, pl.ds(k * 16, 16)] = z16
    wb[0][pl.ds(0, 16)] = z16

    def zcopy(i, carry):
        pltpu.sync_copy(bl[0].at[pl.ds(0, 16)],
                        acc_sh.at[pl.ds(s * RPS + i * 16, 16)])
        pltpu.sync_copy(wb[0].at[pl.ds(0, 16)],
                        den_sh.at[pl.ds(s * RPS + i * 16, 16)])
        return carry

    lax.fori_loop(0, RPS // 16, zcopy, 0, unroll=False)
    plsc.subcore_barrier()

    def decode(slot, ci):
        for g in range(ngr):
            pk = pk_v[pl.ds(ci * ch + g * 16, 16)]
            si[slot][pl.ds(g * 16, 16)] = jnp.minimum(
                lax.shift_right_logical(pk, 14), N - 1)
            di[slot][pl.ds(g * 16, 16)] = jnp.minimum(pk & 0x3FFF, N - 1)
            ab[slot][pl.ds(g * 16, 16)] = jnp.where(
                pk >= VBOUND, -1e30, 0.0)

    def fire_gather(slot):
        pltpu.async_copy(xl_hbm.at[si[slot]], bl[slot], gs[slot])
        pltpu.async_copy(xr_hbm.at[di[slot]], br[slot], gs[slot])

    def drain_gather(slot):
        pltpu.make_async_copy(xl_hbm.at[pl.ds(0, ch)], bl[slot], gs[slot]).wait()
        pltpu.make_async_copy(xl_hbm.at[pl.ds(0, ch)], br[slot], gs[slot]).wait()

    def fire_scatter(slot):
        pltpu.async_copy(bl[slot], acc_sh.at[di[slot]], ss[slot], add=True)
        pltpu.async_copy(wb[slot], den_sh.at[di[slot]], ss[slot], add=True)

    def drain_scatter(slot):
        pltpu.make_async_copy(xl_hbm.at[pl.ds(0, ch)], bl[slot], ss[slot]).wait()
        pltpu.make_async_copy(den_hbm.at[0, pl.ds(0, ch)], wb[slot], ss[slot]).wait()

    def compute(slot):
        blb = bl[slot]
        brb = br[slot]

        def grp(g2, carry2):
            base = g2 * 16
            alpha = jnp.zeros((16,), jnp.float32)
            for e in range(16):
                row = base + e
                part = jnp.zeros((16,), jnp.float32)
                for k in range(nk):
                    v = blb[row, pl.ds(k * 16, 16)] + brb[row, pl.ds(k * 16, 16)]
                    v = jnp.maximum(v, 0.2 * v)
                    part = part + v * att_v[pl.ds(k * 16, 16)]
                for sh in (8, 4, 2, 1):
                    part = part + jnp.take_along_axis(part, lanes ^ sh, axis=0)
                alpha = jnp.where(lanes == e, part, alpha)
            wv = jnp.exp(alpha + ab[slot][pl.ds(base, 16)])
            wb[slot][pl.ds(base, 16)] = wv
            for e in range(16):
                row = base + e
                w = wv[e]
                for k in range(nk):
                    blb[row, pl.ds(k * 16, 16)] = blb[row, pl.ds(k * 16, 16)] * w
            return carry2

        lax.fori_loop(0, ngr, grp, 0, unroll=False)

    for f in range(n - 1):
        decode(f, f)
        fire_gather(f)

    def ring(g, carry):
        for b in range(n):
            ci = g * n + b
            q = (b + n - 1) % n

            @pl.when(ci >= 1)
            def _():
                drain_scatter(q)

            @pl.when(ci + n - 1 < nchunk)
            def _():
                decode(q, ci + n - 1)
                fire_gather(q)

            drain_gather(b)
            compute(b)
            fire_scatter(b)
        return carry

    lax.fori_loop(0, nchunk // n, ring, 0, unroll=False)
    drain_scatter(n - 1)
    plsc.subcore_barrier()
    pltpu.sync_copy(acc_sh.at[pl.ds(s * RPS, RPS)],
                    acc_hbm.at[c, pl.ds(s * RPS, RPS)])
    pltpu.sync_copy(den_sh.at[pl.ds(s * RPS, RPS)],
                    den_hbm.at[c, pl.ds(s * RPS, RPS)])


def _edge_phase(d, untiled, ch, n, xl, xr, edge2, att):
    assert EPT_PAD % ch == 0 and (EPT_PAD // ch) % n == 0 and ch % 16 == 0
    mesh = plsc.VectorSubcoreMesh(core_axis_name="c", subcore_axis_name="s")
    scratch = (
        [pltpu.VMEM((EPT_PAD,), jnp.int32)]
        + [pltpu.VMEM((ch,), jnp.int32)] * (2 * n)
        + [pltpu.VMEM((ch,), jnp.float32)] * n
        + [pltpu.VMEM((ch, d), jnp.float32)] * (2 * n)
        + [pltpu.VMEM((ch,), jnp.float32)] * n
        + [pltpu.VMEM((d,), jnp.float32)]
        + [
            pltpu.VMEM_SHARED((NP, d), jnp.float32),
            pltpu.VMEM_SHARED((NP,), jnp.float32),
        ]
        + [pltpu.SemaphoreType.DMA] * (2 * n)
    )
    params = pltpu.CompilerParams(use_tc_tiling_on_sc=False) if untiled else None
    kfn = pl.kernel(
        functools.partial(_edge_body, d, ch, n),
        out_type=[
            jax.ShapeDtypeStruct((NC, NP, d), jnp.float32),
            jax.ShapeDtypeStruct((NC, NP), jnp.float32),
        ],
        mesh=mesh,
        scratch_types=scratch,
        compiler_params=params,
    )
    return kfn(xl, xr, edge2, att)


def _mid_body(acc_ref, den_ref, b_ref, wl_ref, wr_ref, xl_ref, xr_ref):
    acc = acc_ref[0] + acc_ref[1]
    den = den_ref[0] + den_ref[1] + 1e-16
    h = acc / den + b_ref[...]
    h = jnp.where(h > 0, h, jnp.exp(jnp.minimum(h, 0.0)) - 1.0)
    xl_ref[...] = jnp.dot(h, wl_ref[...], preferred_element_type=jnp.float32)
    xr_ref[...] = jnp.dot(h, wr_ref[...], preferred_element_type=jnp.float32)


def _mid(acc, den3, b1, wl2p, wr2p):
    return pl.pallas_call(
        _mid_body,
        grid=(GRID,),
        in_specs=[
            pl.BlockSpec((NC, ROWS_BLK, D_HID), lambda i: (0, i, 0)),
            pl.BlockSpec((NC, ROWS_BLK, 1), lambda i: (0, i, 0)),
            pl.BlockSpec((1, D_HID), lambda i: (0, 0)),
            pl.BlockSpec((D_HID, NAP), lambda i: (0, 0)),
            pl.BlockSpec((D_HID, NAP), lambda i: (0, 0)),
        ],
        out_specs=[
            pl.BlockSpec((ROWS_BLK, NAP), lambda i: (i, 0)),
            pl.BlockSpec((ROWS_BLK, NAP), lambda i: (i, 0)),
        ],
        out_shape=[
            jax.ShapeDtypeStruct((NP, NAP), jnp.float32),
            jax.ShapeDtypeStruct((NP, NAP), jnp.float32),
        ],
    )(acc, den3, b1.reshape(1, D_HID), wl2p, wr2p)


def _fin_body(acc_ref, den_ref, b_ref, out_ref):
    acc = acc_ref[0] + acc_ref[1]
    den = den_ref[0] + den_ref[1] + 1e-16
    logits = acc / den + b_ref[...]
    lane = lax.broadcasted_iota(jnp.int32, (ROWS_BLK, NAP), 1)
    valid = lane < NA
    neg = jnp.where(valid, logits, -jnp.inf)
    m = jnp.max(neg, axis=1, keepdims=True)
    ex = jnp.where(valid, jnp.exp(logits - m), 0.0)
    se = jnp.sum(ex, axis=1, keepdims=True)
    out_ref[...] = logits - m - jnp.log(se)


def _fin(acc, den3, b2p):
    return pl.pallas_call(
        _fin_body,
        grid=(GRID,),
        in_specs=[
            pl.BlockSpec((NC, ROWS_BLK, NAP), lambda i: (0, i, 0)),
            pl.BlockSpec((NC, ROWS_BLK, 1), lambda i: (0, i, 0)),
            pl.BlockSpec((1, NAP), lambda i: (0, 0)),
        ],
        out_specs=pl.BlockSpec((ROWS_BLK, NAP), lambda i: (i, 0)),
        out_shape=jax.ShapeDtypeStruct((N, NAP), jnp.float32),
    )(acc, den3, b2p.reshape(1, NAP))


@jax.jit
def kernel(x, edge_index, Wl1, Wr1, att1, b1, Wl2, Wr2, att2, b2):
    packed = edge_index[0] * 16384 + edge_index[1]
    pad = jnp.full((NW, EPT_PAD - EPT), PAD_PK, jnp.int32)
    edge2 = jnp.concatenate([packed.reshape(NW, EPT), pad], axis=1)
    wl2p = jnp.pad(Wl2, ((0, 0), (0, NAP - NA)))
    wr2p = jnp.pad(Wr2, ((0, 0), (0, NAP - NA)))
    att2p = jnp.pad(att2, (0, NAP - NA))
    b2p = jnp.pad(b2, (0, NAP - NA))

    xl1, xr1 = _mm2(x, Wl1, Wr1)
    acc1, den1 = _edge_phase(D_HID, False, 32, 3, xl1, xr1, edge2, att1)
    xl2, xr2 = _mid(acc1, den1.reshape(NC, NP, 1), b1, wl2p, wr2p)
    acc2, den2 = _edge_phase(NAP, True, 112, 6, xl2, xr2, edge2, att2p)
    out = _fin(acc2, den2.reshape(NC, NP, 1), b2p)
    return out[:, :NA]


# speedup vs baseline: 1.8173x; 1.0143x over previous
import functools

import jax
import jax.numpy as jnp
from jax import lax
from jax.experimental import pallas as pl
from jax.experimental.pallas import tpu as pltpu
from jax.experimental.pallas import tpu_sc as plsc

N = 10000
E = 320000
D_IN = 128
D_HID = 128
NA = 8
NAP = 16

NC = 2
NS = 16
NW = NC * NS
EPT = E // NW
EPT_PAD = 10080
PAD_PK = 0x0FFFFFFF
VBOUND = N * 16384
NP = 10240
RPS = NP // NS

ROWS_BLK = 1000
GRID = N // ROWS_BLK


def _mm2_body(dt_l, dt_r, x_ref, wl_ref, wr_ref, xl_ref, xr_ref):
    xb = x_ref[...]
    xl_ref[...] = jnp.dot(
        xb, wl_ref[...], preferred_element_type=jnp.float32).astype(dt_l)
    xr_ref[...] = jnp.dot(
        xb, wr_ref[...], preferred_element_type=jnp.float32).astype(dt_r)


def _mm2(x, wl, wr, dt_l=jnp.float32, dt_r=jnp.float32):
    d = wl.shape[1]
    return pl.pallas_call(
        functools.partial(_mm2_body, dt_l, dt_r),
        grid=(GRID,),
        in_specs=[
            pl.BlockSpec((ROWS_BLK, D_IN), lambda i: (i, 0)),
            pl.BlockSpec((D_IN, d), lambda i: (0, 0)),
            pl.BlockSpec((D_IN, d), lambda i: (0, 0)),
        ],
        out_specs=[
            pl.BlockSpec((ROWS_BLK, d), lambda i: (i, 0)),
            pl.BlockSpec((ROWS_BLK, d), lambda i: (i, 0)),
        ],
        out_shape=[
            jax.ShapeDtypeStruct((N, d), dt_l),
            jax.ShapeDtypeStruct((N, d), dt_r),
        ],
    )(x, wl, wr)


def _edge_body(d, ch, n, bf, *refs):
    (xl_hbm, xr_hbm, edge_hbm, att_hbm, acc_hbm, den_hbm, pk_v) = refs[:7]
    si = refs[7:7 + n]
    di = refs[7 + n:7 + 2 * n]
    ab = refs[7 + 2 * n:7 + 3 * n]
    bl = refs[7 + 3 * n:7 + 4 * n]
    br = refs[7 + 4 * n:7 + 5 * n]
    sv = refs[7 + 5 * n:7 + 6 * n]
    wb = refs[7 + 6 * n:7 + 7 * n]
    att_v = refs[7 + 7 * n]
    acc_sh, den_sh = refs[8 + 7 * n:10 + 7 * n]
    gs = refs[10 + 7 * n:10 + 8 * n]
    ss = refs[10 + 8 * n:10 + 9 * n]
    c = lax.axis_index("c")
    s = lax.axis_index("s")
    wid = c * NS + s
    nk = d // 16
    ngr = ch // 16
    nchunk = EPT_PAD // ch
    z16 = jnp.zeros((16,), jnp.float32)
    lanes = lax.iota(jnp.int32, 16)

    pltpu.sync_copy(edge_hbm.at[wid], pk_v)
    pltpu.sync_copy(att_hbm, att_v)

    for e in range(16):
        for k in range(nk):
            sv[0][e, pl.ds(k * 16, 16)] = z16
    wb[0][pl.ds(0, 16)] = z16

    def zcopy(i, carry):
        pltpu.sync_copy(sv[0].at[pl.ds(0, 16)],
                        acc_sh.at[pl.ds(s * RPS + i * 16, 16)])
        pltpu.sync_copy(wb[0].at[pl.ds(0, 16)],
                        den_sh.at[pl.ds(s * RPS + i * 16, 16)])
        return carry

    lax.fori_loop(0, RPS // 16, zcopy, 0, unroll=False)
    plsc.subcore_barrier()

    def decode(slot, ci):
        for g in range(ngr):
            pk = pk_v[pl.ds(ci * ch + g * 16, 16)]
            si[slot][pl.ds(g * 16, 16)] = jnp.minimum(
                lax.shift_right_logical(pk, 14), N - 1)
            di[slot][pl.ds(g * 16, 16)] = jnp.minimum(pk & 0x3FFF, N - 1)
            ab[slot][pl.ds(g * 16, 16)] = jnp.where(
                pk >= VBOUND, -1e30, 0.0)

    def fire_gather(slot):
        pltpu.async_copy(xl_hbm.at[si[slot]], bl[slot], gs[slot])
        pltpu.async_copy(xr_hbm.at[di[slot]], br[slot], gs[slot])

    def drain_gather(slot):
        pltpu.make_async_copy(xl_hbm.at[pl.ds(0, ch)], bl[slot], gs[slot]).wait()
        pltpu.make_async_copy(xr_hbm.at[pl.ds(0, ch)], br[slot], gs[slot]).wait()

    def fire_scatter(slot):
        pltpu.async_copy(sv[slot], acc_sh.at[di[slot]], ss[slot], add=True)
        pltpu.async_copy(wb[slot], den_sh.at[di[slot]], ss[slot], add=True)

    def drain_scatter(slot):
        pltpu.make_async_copy(acc_hbm.at[0, pl.ds(0, ch)], sv[slot], ss[slot]).wait()
        pltpu.make_async_copy(den_hbm.at[0, pl.ds(0, ch)], wb[slot], ss[slot]).wait()

    def compute(slot):
        blb = bl[slot]
        brb = br[slot]
        svb = sv[slot]

        def halves(buf, row, p):
            return plsc.unpack(buf[row, pl.ds(p * 32, 32)],
                               format=plsc.PackFormat.INTERLEAVED,
                               preferred_element_type=jnp.float32)

        def grp(g2, carry2):
            base = g2 * 16
            alpha = jnp.zeros((16,), jnp.float32)
            for e in range(16):
                row = base + e
                part = jnp.zeros((16,), jnp.float32)
                if bf:
                    for p in range(d // 32):
                        re, ro = halves(brb, row, p)
                        v = blb[row, pl.ds(p * 32, 16)] + re
                        v = jnp.maximum(v, 0.2 * v)
                        part = part + v * att_v[pl.ds(p * 32, 16)]
                        v = blb[row, pl.ds(p * 32 + 16, 16)] + ro
                        v = jnp.maximum(v, 0.2 * v)
                        part = part + v * att_v[pl.ds(p * 32 + 16, 16)]
                else:
                    for k in range(nk):
                        v = blb[row, pl.ds(k * 16, 16)] + brb[row, pl.ds(k * 16, 16)]
                        v = jnp.maximum(v, 0.2 * v)
                        part = part + v * att_v[pl.ds(k * 16, 16)]
                for sh in (8, 4, 2, 1):
                    part = part + jnp.take_along_axis(part, lanes ^ sh, axis=0)
                alpha = jnp.where(lanes == e, part, alpha)
            wv = jnp.exp(alpha + ab[slot][pl.ds(base, 16)])
            wb[slot][pl.ds(base, 16)] = wv
            for e in range(16):
                row = base + e
                w = wv[e]
                for k in range(nk):
                    svb[row, pl.ds(k * 16, 16)] = (
                        blb[row, pl.ds(k * 16, 16)] * w)
            return carry2

        lax.fori_loop(0, ngr, grp, 0, unroll=False)

    for f in range(n - 1):
        decode(f, f)
        fire_gather(f)

    def ring(g, carry):
        for b in range(n):
            ci = g * n + b
            q = (b + n - 1) % n

            @pl.when(ci >= 1)
            def _():
                drain_scatter(q)

            @pl.when(ci + n - 1 < nchunk)
            def _():
                decode(q, ci + n - 1)
                fire_gather(q)

            drain_gather(b)
            compute(b)
            fire_scatter(b)
        return carry

    lax.fori_loop(0, nchunk // n, ring, 0, unroll=False)
    drain_scatter(n - 1)
    plsc.subcore_barrier()
    pltpu.sync_copy(acc_sh.at[pl.ds(s * RPS, RPS)],
                    acc_hbm.at[c, pl.ds(s * RPS, RPS)])
    pltpu.sync_copy(den_sh.at[pl.ds(s * RPS, RPS)],
                    den_hbm.at[c, pl.ds(s * RPS, RPS)])


def _edge_phase(d, ch, n, xl, xr, edge2, att):
    assert EPT_PAD % ch == 0 and (EPT_PAD // ch) % n == 0 and ch % 16 == 0
    bf = xr.dtype == jnp.bfloat16
    mesh = plsc.VectorSubcoreMesh(core_axis_name="c", subcore_axis_name="s")
    scratch = (
        [pltpu.VMEM((EPT_PAD,), jnp.int32)]
        + [pltpu.VMEM((ch,), jnp.int32)] * (2 * n)
        + [pltpu.VMEM((ch,), jnp.float32)] * n
        + [pltpu.VMEM((ch, d), xl.dtype)] * n
        + [pltpu.VMEM((ch, d), xr.dtype)] * n
        + [pltpu.VMEM((ch, d), jnp.float32)] * n
        + [pltpu.VMEM((ch,), jnp.float32)] * n
        + [pltpu.VMEM((d,), jnp.float32)]
        + [
            pltpu.VMEM_SHARED((NP, d), jnp.float32),
            pltpu.VMEM_SHARED((NP,), jnp.float32),
        ]
        + [pltpu.SemaphoreType.DMA] * (2 * n)
    )
    params = pltpu.CompilerParams(use_tc_tiling_on_sc=False, needs_layout_passes=False)
    kfn = pl.kernel(
        functools.partial(_edge_body, d, ch, n, bf),
        out_type=[
            jax.ShapeDtypeStruct((NC, NP, d), jnp.float32),
            jax.ShapeDtypeStruct((NC, NP), jnp.float32),
        ],
        mesh=mesh,
        scratch_types=scratch,
        compiler_params=params,
    )
    return kfn(xl, xr, edge2, att)


def _mid_body(acc_ref, den_ref, b_ref, wl_ref, wr_ref, xl_ref, xr_ref):
    acc = acc_ref[0] + acc_ref[1]
    den = den_ref[0] + den_ref[1] + 1e-16
    h = acc / den + b_ref[...]
    h = jnp.where(h > 0, h, jnp.exp(jnp.minimum(h, 0.0)) - 1.0)
    xl_ref[...] = jnp.dot(h, wl_ref[...], preferred_element_type=jnp.float32)
    xr_ref[...] = jnp.dot(h, wr_ref[...], preferred_element_type=jnp.float32)


def _mid(acc, den3, b1, wl2p, wr2p):
    return pl.pallas_call(
        _mid_body,
        grid=(GRID,),
        in_specs=[
            pl.BlockSpec((NC, ROWS_BLK, D_HID), lambda i: (0, i, 0)),
            pl.BlockSpec((NC, ROWS_BLK, 1), lambda i: (0, i, 0)),
            pl.BlockSpec((1, D_HID), lambda i: (0, 0)),
            pl.BlockSpec((D_HID, NAP), lambda i: (0, 0)),
            pl.BlockSpec((D_HID, NAP), lambda i: (0, 0)),
        ],
        out_specs=[
            pl.BlockSpec((ROWS_BLK, NAP), lambda i: (i, 0)),
            pl.BlockSpec((ROWS_BLK, NAP), lambda i: (i, 0)),
        ],
        out_shape=[
            jax.ShapeDtypeStruct((NP, NAP), jnp.float32),
            jax.ShapeDtypeStruct((NP, NAP), jnp.float32),
        ],
    )(acc, den3, b1.reshape(1, D_HID), wl2p, wr2p)


def _fin_body(acc_ref, den_ref, b_ref, out_ref):
    acc = acc_ref[0] + acc_ref[1]
    den = den_ref[0] + den_ref[1] + 1e-16
    logits = acc / den + b_ref[...]
    lane = lax.broadcasted_iota(jnp.int32, (ROWS_BLK, NAP), 1)
    valid = lane < NA
    neg = jnp.where(valid, logits, -jnp.inf)
    m = jnp.max(neg, axis=1, keepdims=True)
    ex = jnp.where(valid, jnp.exp(logits - m), 0.0)
    se = jnp.sum(ex, axis=1, keepdims=True)
    out_ref[...] = logits - m - jnp.log(se)


def _fin(acc, den3, b2p):
    return pl.pallas_call(
        _fin_body,
        grid=(GRID,),
        in_specs=[
            pl.BlockSpec((NC, ROWS_BLK, NAP), lambda i: (0, i, 0)),
            pl.BlockSpec((NC, ROWS_BLK, 1), lambda i: (0, i, 0)),
            pl.BlockSpec((1, NAP), lambda i: (0, 0)),
        ],
        out_specs=pl.BlockSpec((ROWS_BLK, NAP), lambda i: (i, 0)),
        out_shape=jax.ShapeDtypeStruct((N, NAP), jnp.float32),
    )(acc, den3, b2p.reshape(1, NAP))


@jax.jit
def kernel(x, edge_index, Wl1, Wr1, att1, b1, Wl2, Wr2, att2, b2):
    packed = edge_index[0] * 16384 + edge_index[1]
    pad = jnp.full((NW, EPT_PAD - EPT), PAD_PK, jnp.int32)
    edge2 = jnp.concatenate([packed.reshape(NW, EPT), pad], axis=1)
    wl2p = jnp.pad(Wl2, ((0, 0), (0, NAP - NA)))
    wr2p = jnp.pad(Wr2, ((0, 0), (0, NAP - NA)))
    att2p = jnp.pad(att2, (0, NAP - NA))
    b2p = jnp.pad(b2, (0, NAP - NA))

    i16 = jnp.arange(16, dtype=jnp.int32)
    qblk = jnp.stack([i16, i16 + 16], axis=1).reshape(32)
    qidx = jnp.concatenate([qblk + 32 * p for p in range(D_HID // 32)])
    xl1, xr1 = _mm2(x, Wl1, Wr1[:, qidx], jnp.float32, jnp.bfloat16)
    acc1, den1 = _edge_phase(D_HID, 32, 3, xl1, xr1, edge2, att1)
    xl2, xr2 = _mid(acc1, den1.reshape(NC, NP, 1), b1, wl2p, wr2p)
    acc2, den2 = _edge_phase(NAP, 112, 3, xl2, xr2, edge2, att2p)
    out = _fin(acc2, den2.reshape(NC, NP, 1), b2p)
    return out[:, :NA]
